# baseline (speedup 1.0000x reference)
import functools
import numpy as np
import jax
import jax.numpy as jnp
from jax.experimental import pallas as pl
from jax.experimental.pallas import tpu as pltpu

N = 10000
E = 320000
D = 128
NRBF = 16
CUTOFF = 5.0

EB = 512
NB = 2000

_CENTERS = np.linspace(0.5, CUTOFF, NRBF, dtype=np.float32)


def _embed_body(an_ref, embed_ref, h_ref):
    an = an_ref[...]
    z = jax.lax.broadcasted_iota(jnp.int32, (1, 128), 1)
    onehot = (an == z).astype(jnp.float32)
    h_ref[...] = jnp.dot(onehot, embed_ref[...],
                         preferred_element_type=jnp.float32)


def _rbf_body(r_ref, wrbf_ref, fm_ref, rp_ref):
    r = r_ref[...]
    step = (CUTOFF - 0.5) / (NRBF - 1)
    centers = 0.5 + step * jax.lax.broadcasted_iota(
        jnp.int32, (1, NRBF), 1).astype(jnp.float32)
    dkr = r - centers
    g = jnp.exp(-2.0 * dkr * dkr)
    x = r / CUTOFF
    inside = x < 1.0
    env = jnp.where(inside, 0.5 * (jnp.cos(jnp.pi * x) + 1.0), 0.0)
    envp = jnp.where(inside, -(jnp.pi / (2.0 * CUTOFF)) * jnp.sin(jnp.pi * x), 0.0)
    R = g * env
    rp_ref[...] = (-4.0 * dkr * g) * env + g * envp
    fm_ref[...] = jnp.dot(R, wrbf_ref[...], preferred_element_type=jnp.float32)


def _node_body(h_ref, agg_ref, w0_ref, w1_ref, wout_ref, b_ref):
    h = h_ref[...]
    agg = agg_ref[...]
    u = (jnp.dot(h, w0_ref[...], preferred_element_type=jnp.float32)
         + jnp.dot(agg, w1_ref[...], preferred_element_type=jnp.float32))
    sig = jax.nn.sigmoid(u)
    silup = sig * (1.0 + u * (1.0 - sig))
    a = silup * wout_ref[...]
    b_ref[...] = jax.lax.dot_general(
        a, w1_ref[...], (((1,), (1,)), ((), ())),
        preferred_element_type=jnp.float32)


def _edge_back_body(hs_ref, bd_ref, rp_ref, wrbf_ref, ti_ref):
    s = hs_ref[...] * bd_ref[...]
    q = jax.lax.dot_general(
        s, wrbf_ref[...], (((1,), (1,)), ((), ())),
        preferred_element_type=jnp.float32)
    t = jnp.sum(q * rp_ref[...], axis=1, keepdims=True)
    ti_ref[...] = t


def kernel(positions, cell, shifts_idx, edge_index, atomic_numbers, embed, W_rbf, W0, W1, w_out):
    src = edge_index[0].astype(jnp.int32)
    dst = edge_index[1].astype(jnp.int32)
    an = atomic_numbers.astype(jnp.int32)

    embed_pad = jnp.zeros((128, 128), jnp.float32).at[:embed.shape[0]].set(embed)
    h = pl.pallas_call(
        _embed_body,
        grid=(N // NB,),
        in_specs=[pl.BlockSpec((NB, 1), lambda i: (i, 0)),
                  pl.BlockSpec((128, 128), lambda i: (0, 0))],
        out_specs=pl.BlockSpec((NB, 128), lambda i: (i, 0)),
        out_shape=jax.ShapeDtypeStruct((N, 128), jnp.float32),
    )(an[:, None], embed_pad)

    v = positions[dst] - positions[src]
    r2 = jnp.sum(v * v, axis=-1)
    r = jnp.sqrt(r2 + 1e-12)
    invr = 1.0 / r

    Fm, Rp = pl.pallas_call(
        _rbf_body,
        grid=(E // EB,),
        in_specs=[pl.BlockSpec((EB, 1), lambda i: (i, 0)),
                  pl.BlockSpec((NRBF, 128), lambda i: (0, 0))],
        out_specs=[pl.BlockSpec((EB, 128), lambda i: (i, 0)),
                   pl.BlockSpec((EB, NRBF), lambda i: (i, 0))],
        out_shape=[jax.ShapeDtypeStruct((E, 128), jnp.float32),
                   jax.ShapeDtypeStruct((E, NRBF), jnp.float32)],
    )(r[:, None], W_rbf)

    hs = h[src]
    msg = hs * Fm
    agg = jax.ops.segment_sum(msg, dst, num_segments=N)

    b = pl.pallas_call(
        _node_body,
        grid=(N // NB,),
        in_specs=[pl.BlockSpec((NB, 128), lambda i: (i, 0)),
                  pl.BlockSpec((NB, 128), lambda i: (i, 0)),
                  pl.BlockSpec((128, 128), lambda i: (0, 0)),
                  pl.BlockSpec((128, 128), lambda i: (0, 0)),
                  pl.BlockSpec((1, 128), lambda i: (0, 0))],
        out_specs=pl.BlockSpec((NB, 128), lambda i: (i, 0)),
        out_shape=jax.ShapeDtypeStruct((N, 128), jnp.float32),
    )(h, agg, W0, W1, w_out[None, :])

    bd = b[dst]
    ti = pl.pallas_call(
        _edge_back_body,
        grid=(E // EB,),
        in_specs=[pl.BlockSpec((EB, 128), lambda i: (i, 0)),
                  pl.BlockSpec((EB, 128), lambda i: (i, 0)),
                  pl.BlockSpec((EB, NRBF), lambda i: (i, 0)),
                  pl.BlockSpec((NRBF, 128), lambda i: (0, 0))],
        out_specs=pl.BlockSpec((EB, 1), lambda i: (i, 0)),
        out_shape=jax.ShapeDtypeStruct((E, 1), jnp.float32),
    )(hs, bd, Rp, W_rbf)

    w = (ti[:, 0] * invr)[:, None] * v
    forces = (jax.ops.segment_sum(w, src, num_segments=N)
              - jax.ops.segment_sum(w, dst, num_segments=N))
    return forces


# speedup vs baseline: 1.1373x; 1.1373x over previous
import functools
import numpy as np
import jax
import jax.numpy as jnp
from jax import lax
from jax.experimental import pallas as pl
from jax.experimental.pallas import tpu as pltpu
from jax.experimental.pallas import tpu_sc as plsc

N = 10000
E = 320000
D = 128
NRBF = 16
CUTOFF = 5.0

EB = 512
NB = 2000

_CENTERS = np.linspace(0.5, CUTOFF, NRBF, dtype=np.float32)

NCORE = 2
NSUB = 16
NWORK = NCORE * NSUB
EPW = E // NWORK
ECH = 80
NPAD = 10240
NROWS = NPAD // NSUB

_sc_mesh = plsc.VectorSubcoreMesh(core_axis_name="c", subcore_axis_name="s")


@functools.partial(
    pl.kernel, mesh=_sc_mesh,
    out_type=jax.ShapeDtypeStruct((NCORE, NPAD, D), jnp.float32),
    scratch_types=[
        pltpu.VMEM((ECH,), jnp.int32),
        pltpu.VMEM((ECH,), jnp.int32),
        pltpu.VMEM((ECH, D), jnp.float32),
        pltpu.VMEM((ECH, D), jnp.float32),
        pltpu.VMEM_SHARED((NPAD, D), jnp.float32),
        pltpu.SemaphoreType.DMA,
    ],
)
def _fwd_edge_sc(h_hbm, fm_hbm, src_hbm, dst_hbm, zeros_hbm, agg_hbm,
                 src_v, dst_v, rows_v, fm_v, agg_sp, gsem):
    c = lax.axis_index("c")
    s = lax.axis_index("s")
    wid = s * NCORE + c
    base = wid * EPW

    pltpu.sync_copy(zeros_hbm, agg_sp.at[pl.ds(s * NROWS, NROWS)])
    plsc.subcore_barrier()

    def chunk(i, carry):
        off = base + i * ECH
        pltpu.sync_copy(src_hbm.at[pl.ds(off, ECH)], src_v)
        pltpu.sync_copy(dst_hbm.at[pl.ds(off, ECH)], dst_v)
        pltpu.async_copy(h_hbm.at[src_v], rows_v, gsem).wait()
        pltpu.sync_copy(fm_hbm.at[pl.ds(off, ECH)], fm_v)

        def mult(j, carry2):
            for k in range(D // 16):
                sl = pl.ds(k * 16, 16)
                rows_v[j, sl] = rows_v[j, sl] * fm_v[j, sl]
            return carry2
        lax.fori_loop(0, ECH, mult, 0)

        pltpu.sync_copy(rows_v, agg_sp.at[dst_v], add=True)
        return carry
    lax.fori_loop(0, EPW // ECH, chunk, 0)

    plsc.subcore_barrier()
    sl = pl.ds(s * NROWS, NROWS)
    pltpu.sync_copy(agg_sp.at[sl], agg_hbm.at[c, sl])


def _embed_body(an_ref, embed_ref, h_ref):
    an = an_ref[...]
    z = jax.lax.broadcasted_iota(jnp.int32, (1, 128), 1)
    onehot = (an == z).astype(jnp.float32)
    h_ref[...] = jnp.dot(onehot, embed_ref[...],
                         preferred_element_type=jnp.float32)


def _rbf_body(r_ref, wrbf_ref, fm_ref, rp_ref):
    r = r_ref[...]
    step = (CUTOFF - 0.5) / (NRBF - 1)
    centers = 0.5 + step * jax.lax.broadcasted_iota(
        jnp.int32, (1, NRBF), 1).astype(jnp.float32)
    dkr = r - centers
    g = jnp.exp(-2.0 * dkr * dkr)
    x = r / CUTOFF
    inside = x < 1.0
    env = jnp.where(inside, 0.5 * (jnp.cos(jnp.pi * x) + 1.0), 0.0)
    envp = jnp.where(inside, -(jnp.pi / (2.0 * CUTOFF)) * jnp.sin(jnp.pi * x), 0.0)
    R = g * env
    rp_ref[...] = (-4.0 * dkr * g) * env + g * envp
    fm_ref[...] = jnp.dot(R, wrbf_ref[...], preferred_element_type=jnp.float32)


def _node_body(h_ref, agg_ref, w0_ref, w1_ref, wout_ref, b_ref):
    h = h_ref[...]
    agg = agg_ref[...]
    u = (jnp.dot(h, w0_ref[...], preferred_element_type=jnp.float32)
         + jnp.dot(agg, w1_ref[...], preferred_element_type=jnp.float32))
    sig = jax.nn.sigmoid(u)
    silup = sig * (1.0 + u * (1.0 - sig))
    a = silup * wout_ref[...]
    b_ref[...] = jax.lax.dot_general(
        a, w1_ref[...], (((1,), (1,)), ((), ())),
        preferred_element_type=jnp.float32)


def _edge_back_body(hs_ref, bd_ref, rp_ref, wrbf_ref, ti_ref):
    s = hs_ref[...] * bd_ref[...]
    q = jax.lax.dot_general(
        s, wrbf_ref[...], (((1,), (1,)), ((), ())),
        preferred_element_type=jnp.float32)
    t = jnp.sum(q * rp_ref[...], axis=1, keepdims=True)
    ti_ref[...] = t


def kernel(positions, cell, shifts_idx, edge_index, atomic_numbers, embed, W_rbf, W0, W1, w_out):
    src = edge_index[0].astype(jnp.int32)
    dst = edge_index[1].astype(jnp.int32)
    an = atomic_numbers.astype(jnp.int32)

    embed_pad = jnp.zeros((128, 128), jnp.float32).at[:embed.shape[0]].set(embed)
    h = pl.pallas_call(
        _embed_body,
        grid=(N // NB,),
        in_specs=[pl.BlockSpec((NB, 1), lambda i: (i, 0)),
                  pl.BlockSpec((128, 128), lambda i: (0, 0))],
        out_specs=pl.BlockSpec((NB, 128), lambda i: (i, 0)),
        out_shape=jax.ShapeDtypeStruct((N, 128), jnp.float32),
    )(an[:, None], embed_pad)

    v = positions[dst] - positions[src]
    r2 = jnp.sum(v * v, axis=-1)
    r = jnp.sqrt(r2 + 1e-12)
    invr = 1.0 / r

    Fm, Rp = pl.pallas_call(
        _rbf_body,
        grid=(E // EB,),
        in_specs=[pl.BlockSpec((EB, 1), lambda i: (i, 0)),
                  pl.BlockSpec((NRBF, 128), lambda i: (0, 0))],
        out_specs=[pl.BlockSpec((EB, 128), lambda i: (i, 0)),
                   pl.BlockSpec((EB, NRBF), lambda i: (i, 0))],
        out_shape=[jax.ShapeDtypeStruct((E, 128), jnp.float32),
                   jax.ShapeDtypeStruct((E, NRBF), jnp.float32)],
    )(r[:, None], W_rbf)

    zeros_rows = jnp.zeros((NROWS, D), jnp.float32)
    agg2 = _fwd_edge_sc(h, Fm, src, dst, zeros_rows)
    agg = agg2[0, :N] + agg2[1, :N]
    hs = h[src]

    b = pl.pallas_call(
        _node_body,
        grid=(N // NB,),
        in_specs=[pl.BlockSpec((NB, 128), lambda i: (i, 0)),
                  pl.BlockSpec((NB, 128), lambda i: (i, 0)),
                  pl.BlockSpec((128, 128), lambda i: (0, 0)),
                  pl.BlockSpec((128, 128), lambda i: (0, 0)),
                  pl.BlockSpec((1, 128), lambda i: (0, 0))],
        out_specs=pl.BlockSpec((NB, 128), lambda i: (i, 0)),
        out_shape=jax.ShapeDtypeStruct((N, 128), jnp.float32),
    )(h, agg, W0, W1, w_out[None, :])

    bd = b[dst]
    ti = pl.pallas_call(
        _edge_back_body,
        grid=(E // EB,),
        in_specs=[pl.BlockSpec((EB, 128), lambda i: (i, 0)),
                  pl.BlockSpec((EB, 128), lambda i: (i, 0)),
                  pl.BlockSpec((EB, NRBF), lambda i: (i, 0)),
                  pl.BlockSpec((NRBF, 128), lambda i: (0, 0))],
        out_specs=pl.BlockSpec((EB, 1), lambda i: (i, 0)),
        out_shape=jax.ShapeDtypeStruct((E, 1), jnp.float32),
    )(hs, bd, Rp, W_rbf)

    w = (ti[:, 0] * invr)[:, None] * v
    forces = (jax.ops.segment_sum(w, src, num_segments=N)
              - jax.ops.segment_sum(w, dst, num_segments=N))
    return forces
